# baseline (speedup 1.0000x reference)
import functools

import jax
import jax.numpy as jnp
from jax import lax
from jax.experimental import pallas as pl
from jax.experimental.pallas import tpu as pltpu
from jax.experimental.pallas import tpu_sc as plsc

N_NODES = 10000
E = 320000
D = 128
HID = 16
NREL = 5
G = E // 2

NC, NS = 2, 16
NW = NC * NS
GPW = G // NW
CHP = 40
NCHUNK = GPW // CHP
NBUF = 5
NJ = NCHUNK // NBUF

BT2 = 4000
NBT = G // BT2

HIMASK = -65536


def _sc_gather_mul(x32, row, col):
    mesh = plsc.VectorSubcoreMesh(
        core_axis_name="c", subcore_axis_name="s", num_cores=NC, num_subcores=NS)

    @functools.partial(
        pl.kernel,
        out_type=jax.ShapeDtypeStruct((G, D), jnp.int32),
        mesh=mesh,
        compiler_params=pltpu.CompilerParams(
            needs_layout_passes=False, use_tc_tiling_on_sc=False),
        scratch_types=[
            [pltpu.VMEM((GPW,), jnp.int32) for _ in range(4)],
            [[pltpu.VMEM((CHP, D // 2), jnp.int32) for _ in range(4)]
             for _ in range(NBUF)],
            [pltpu.VMEM((CHP, D), jnp.int32) for _ in range(NBUF)],
            [[pltpu.SemaphoreType.DMA for _ in range(4)]
             for _ in range(NBUF)],
            [pltpu.SemaphoreType.DMA for _ in range(NBUF)],
        ],
    )
    def k(x_hbm, row_hbm, col_hbm, y_hbm, idx, xb, yb, smg, sst):
        wid = lax.axis_index("s") * NC + lax.axis_index("c")
        gbase = pl.multiple_of(wid * GPW, GPW)
        for side in range(2):
            pltpu.sync_copy(row_hbm.at[pl.ds(side * G + gbase, GPW)],
                            idx[2 * side])
            pltpu.sync_copy(col_hbm.at[pl.ds(side * G + gbase, GPW)],
                            idx[2 * side + 1])

        def fire(ci, b):
            off = pl.multiple_of(ci * CHP, CHP)
            for q in range(4):
                pltpu.async_copy(x_hbm.at[idx[q].at[pl.ds(off, CHP)]],
                                 xb[b][q], smg[b][q])

        def gwait(b):
            for q in range(4):
                pltpu.make_async_copy(
                    x_hbm.at[idx[q].at[pl.ds(0, CHP)]], xb[b][q],
                    smg[b][q]).wait()

        def swait(b):
            pltpu.make_async_copy(yb[b], y_hbm.at[pl.ds(0, CHP)],
                                  sst[b]).wait()

        for b in range(NBUF - 1):
            fire(b, b)

        def outer(j, carry):
            for b in range(NBUF):
                ci = j * NBUF + b
                gwait(b)

                def rowmul(r2, c2):
                    for side in range(2):
                        for kk in range(D // 32):
                            s = pl.ds(kk * 16, 16)
                            vi = xb[b][2 * side][r2, s]
                            vj = xb[b][2 * side + 1][r2, s]
                            a0 = plsc.bitcast(vi << 16, jnp.float32)
                            b0 = plsc.bitcast(vj << 16, jnp.float32)
                            a1 = plsc.bitcast(vi & HIMASK, jnp.float32)
                            b1 = plsc.bitcast(vj & HIMASK, jnp.float32)
                            p0 = plsc.bitcast(a0 * b0, jnp.int32)
                            p1 = plsc.bitcast(a1 * b1, jnp.int32)
                            q0 = lax.shift_right_logical(p0 + 0x8000, 16)
                            q1 = (p1 + 0x8000) & HIMASK
                            yb[b][r2, pl.ds(side * 64 + kk * 16, 16)] = (
                                q0 | q1)
                    return c2

                lax.fori_loop(0, CHP, rowmul, 0, unroll=2)
                off = pl.multiple_of(ci * CHP, CHP)
                pltpu.async_copy(yb[b], y_hbm.at[pl.ds(gbase + off, CHP)],
                                 sst[b])
                nb = (b + NBUF - 1) % NBUF
                if b == 0:
                    @pl.when(j > 0)
                    def _():
                        swait(nb)
                    fire(ci + NBUF - 1, nb)
                else:
                    @pl.when(j < NJ - 1)
                    def _():
                        swait(nb)
                        fire(ci + NBUF - 1, nb)
            return carry

        lax.fori_loop(0, NJ, outer, 0)
        for b in range(NBUF):
            swait(b)

    return k(x32, row, col)


def _tc_mlp(y32, concs, rel2d, W0a, wc, b0r, Wf, bf):
    def body(y_ref, cA_ref, cB_ref, rA_ref, rB_ref, W0a_ref, wc_ref, b0_ref,
             Wf_ref, bf_ref, ones_ref, oA_ref, oB_ref):
        v = y_ref[...]
        ylo = lax.bitcast_convert_type(v << 16, jnp.float32)
        yhi = lax.bitcast_convert_type(v & HIMASK, jnp.float32)
        ylo = ylo.astype(jnp.bfloat16)
        yhi = yhi.astype(jnp.bfloat16)
        for half, c_ref, r_ref, o_ref in ((0, cA_ref, rA_ref, oA_ref),
                                          (1, cB_ref, rB_ref, oB_ref)):
            sl = slice(half * 64, half * 64 + 64)
            yh = jnp.concatenate([ylo[:, sl], yhi[:, sl]], axis=1)
            cc = c_ref[...]
            c = cc[:, 0:1] * cc[:, 1:2]
            h = jnp.dot(yh, W0a_ref[...], preferred_element_type=jnp.float32)
            h = jnp.maximum(h + c * wc_ref[...] + b0_ref[...], 0.0)
            o5 = jnp.dot(h, Wf_ref[...], preferred_element_type=jnp.float32)
            o5 = o5 + bf_ref[...]
            onehot = (r_ref[...]
                      == lax.broadcasted_iota(jnp.int32, (1, NREL), 1))
            sel = o5 * onehot.astype(jnp.float32)
            o_ref[...] = jnp.dot(sel, ones_ref[...],
                                 preferred_element_type=jnp.float32)

    ones5 = jnp.ones((NREL, 1), jnp.float32)
    return pl.pallas_call(
        body,
        grid=(NBT,),
        in_specs=[
            pl.BlockSpec((BT2, D), lambda i: (i, 0)),
            pl.BlockSpec((BT2, 2), lambda i: (i, 0)),
            pl.BlockSpec((BT2, 2), lambda i: (i + NBT, 0)),
            pl.BlockSpec((BT2, 1), lambda i: (i, 0)),
            pl.BlockSpec((BT2, 1), lambda i: (i + NBT, 0)),
            pl.BlockSpec((D, HID), lambda i: (0, 0)),
            pl.BlockSpec((1, HID), lambda i: (0, 0)),
            pl.BlockSpec((1, HID), lambda i: (0, 0)),
            pl.BlockSpec((HID, NREL), lambda i: (0, 0)),
            pl.BlockSpec((1, NREL), lambda i: (0, 0)),
            pl.BlockSpec((NREL, 1), lambda i: (0, 0)),
        ],
        out_specs=[pl.BlockSpec((BT2, 1), lambda i: (i, 0)),
                   pl.BlockSpec((BT2, 1), lambda i: (i, 0))],
        out_shape=[jax.ShapeDtypeStruct((G, 1), jnp.float32),
                   jax.ShapeDtypeStruct((G, 1), jnp.float32)],
    )(y32, concs, concs, rel2d, rel2d, W0a, wc, b0r, Wf, bf, ones5)


def kernel(x, edge_index, relations, concs, W0, b0, Wr, br):
    xb = lax.bitcast_convert_type(x, jnp.int32)
    lo = lax.shift_right_logical(xb[:, 0::2] + 0x8000, 16)
    hi = (xb[:, 1::2] + 0x8000) & HIMASK
    x32 = lo | hi
    y32 = _sc_gather_mul(x32, edge_index[:, 0], edge_index[:, 1])
    W0ab = W0[:D].astype(jnp.bfloat16)
    W0a = jnp.concatenate([W0ab[0::2], W0ab[1::2]], axis=0)
    wc = W0[D:D + 1, :]
    Wf = Wr[:, :, 0].T
    bf = br[:, 0][None, :]
    oA, oB = _tc_mlp(y32, concs, relations[:, None], W0a, wc, b0[None, :],
                     Wf, bf)
    return jnp.concatenate([oA, oB], axis=0)


# speedup vs baseline: 1.0116x; 1.0116x over previous
import functools

import jax
import jax.numpy as jnp
from jax import lax
from jax.experimental import pallas as pl
from jax.experimental.pallas import tpu as pltpu
from jax.experimental.pallas import tpu_sc as plsc

N_NODES = 10000
E = 320000
D = 128
HID = 16
NREL = 5
G = E // 2

NC, NS = 2, 16
NW = NC * NS
GPW = G // NW
CHP = 40
NCHUNK = GPW // CHP
NBUF = 5
NJ = NCHUNK // NBUF

BT2 = 4000
NBT = G // BT2

BPK = 2000

HIMASK = -65536


def _pack_table(x, se, so):
    def body(x_ref, se_ref, so_ref, o_ref):
        xe = jnp.dot(x_ref[...], se_ref[...],
                     preferred_element_type=jnp.float32)
        xo = jnp.dot(x_ref[...], so_ref[...],
                     preferred_element_type=jnp.float32)
        pe = lax.bitcast_convert_type(xe, jnp.int32)
        po = lax.bitcast_convert_type(xo, jnp.int32)
        lo = lax.shift_right_logical(pe + 0x8000, 16)
        hi = (po + 0x8000) & HIMASK
        o_ref[...] = lo | hi

    return pl.pallas_call(
        body,
        grid=(N_NODES // BPK,),
        in_specs=[
            pl.BlockSpec((BPK, D), lambda i: (i, 0)),
            pl.BlockSpec((D, D // 2), lambda i: (0, 0)),
            pl.BlockSpec((D, D // 2), lambda i: (0, 0)),
        ],
        out_specs=pl.BlockSpec((BPK, D // 2), lambda i: (i, 0)),
        out_shape=jax.ShapeDtypeStruct((N_NODES, D // 2), jnp.int32),
    )(x, se, so)


def _sc_gather_mul(x32, eflat):
    mesh = plsc.VectorSubcoreMesh(
        core_axis_name="c", subcore_axis_name="s", num_cores=NC, num_subcores=NS)

    @functools.partial(
        pl.kernel,
        out_type=jax.ShapeDtypeStruct((G, D), jnp.int32),
        mesh=mesh,
        compiler_params=pltpu.CompilerParams(
            needs_layout_passes=False, use_tc_tiling_on_sc=False),
        scratch_types=[
            [pltpu.VMEM((2 * GPW,), jnp.int32) for _ in range(2)],
            [[pltpu.VMEM((2 * CHP, D // 2), jnp.int32) for _ in range(2)]
             for _ in range(NBUF)],
            [pltpu.VMEM((CHP, D), jnp.int32) for _ in range(NBUF)],
            [[pltpu.SemaphoreType.DMA for _ in range(2)]
             for _ in range(NBUF)],
            [pltpu.SemaphoreType.DMA for _ in range(NBUF)],
        ],
    )
    def k(x_hbm, ei_hbm, y_hbm, idx, xb, yb, smg, sst):
        wid = lax.axis_index("s") * NC + lax.axis_index("c")
        gbase = pl.multiple_of(wid * GPW, GPW)
        for side in range(2):
            pltpu.sync_copy(
                ei_hbm.at[pl.ds(2 * (side * G + gbase), 2 * GPW)], idx[side])

        def fire(ci, b):
            off = pl.multiple_of(2 * ci * CHP, 2 * CHP)
            for side in range(2):
                pltpu.async_copy(
                    x_hbm.at[idx[side].at[pl.ds(off, 2 * CHP)]],
                    xb[b][side], smg[b][side])

        def gwait(b):
            for side in range(2):
                pltpu.make_async_copy(
                    x_hbm.at[idx[side].at[pl.ds(0, 2 * CHP)]], xb[b][side],
                    smg[b][side]).wait()

        def swait(b):
            pltpu.make_async_copy(yb[b], y_hbm.at[pl.ds(0, CHP)],
                                  sst[b]).wait()

        for b in range(NBUF - 1):
            fire(b, b)

        def outer(j, carry):
            for b in range(NBUF):
                ci = j * NBUF + b
                gwait(b)

                def rowmul(r2, c2):
                    for side in range(2):
                        for kk in range(D // 32):
                            s = pl.ds(kk * 16, 16)
                            vi = xb[b][side][2 * r2, s]
                            vj = xb[b][side][2 * r2 + 1, s]
                            a0 = plsc.bitcast(vi << 16, jnp.float32)
                            b0 = plsc.bitcast(vj << 16, jnp.float32)
                            a1 = plsc.bitcast(vi & HIMASK, jnp.float32)
                            b1 = plsc.bitcast(vj & HIMASK, jnp.float32)
                            p0 = plsc.bitcast(a0 * b0, jnp.int32)
                            p1 = plsc.bitcast(a1 * b1, jnp.int32)
                            q0 = lax.shift_right_logical(p0 + 0x8000, 16)
                            q1 = (p1 + 0x8000) & HIMASK
                            yb[b][r2, pl.ds(side * 64 + kk * 16, 16)] = (
                                q0 | q1)
                    return c2

                lax.fori_loop(0, CHP, rowmul, 0, unroll=2)
                off = pl.multiple_of(ci * CHP, CHP)
                pltpu.async_copy(yb[b], y_hbm.at[pl.ds(gbase + off, CHP)],
                                 sst[b])
                nb = (b + NBUF - 1) % NBUF
                if b == 0:
                    @pl.when(j > 0)
                    def _():
                        swait(nb)
                    fire(ci + NBUF - 1, nb)
                else:
                    @pl.when(j < NJ - 1)
                    def _():
                        swait(nb)
                        fire(ci + NBUF - 1, nb)
            return carry

        lax.fori_loop(0, NJ, outer, 0)
        for b in range(NBUF):
            swait(b)

    return k(x32, eflat)


def _tc_mlp(y32, concs, relw, W0a, wc, b0r, Wf, bf):
    def body(y_ref, cA_ref, cB_ref, rA_ref, rB_ref, W0a_ref, wc_ref, b0_ref,
             Wf_ref, bf_ref, ones_ref, oA_ref, oB_ref):
        v = y_ref[...]
        ylo = lax.bitcast_convert_type(v << 16, jnp.float32)
        yhi = lax.bitcast_convert_type(v & HIMASK, jnp.float32)
        ylo = ylo.astype(jnp.bfloat16)
        yhi = yhi.astype(jnp.bfloat16)
        for half, c_ref, r_ref, o_ref in ((0, cA_ref, rA_ref, oA_ref),
                                          (1, cB_ref, rB_ref, oB_ref)):
            sl = slice(half * 64, half * 64 + 64)
            yh = jnp.concatenate([ylo[:, sl], yhi[:, sl]], axis=1)
            cc = c_ref[...]
            c = cc[:, 0:1] * cc[:, 1:2]
            h = jnp.dot(yh, W0a_ref[...], preferred_element_type=jnp.float32)
            h = jnp.maximum(h + c * wc_ref[...] + b0_ref[...], 0.0)
            o5 = jnp.dot(h, Wf_ref[...], preferred_element_type=jnp.float32)
            o5 = o5 + bf_ref[...]
            rel = jnp.transpose(r_ref[0])
            onehot = (rel
                      == lax.broadcasted_iota(jnp.int32, (1, NREL), 1))
            sel = o5 * onehot.astype(jnp.float32)
            o_ref[...] = jnp.dot(sel, ones_ref[...],
                                 preferred_element_type=jnp.float32)

    ones5 = jnp.ones((NREL, 1), jnp.float32)
    return pl.pallas_call(
        body,
        grid=(NBT,),
        in_specs=[
            pl.BlockSpec((BT2, D), lambda i: (i, 0)),
            pl.BlockSpec((BT2, 2), lambda i: (i, 0)),
            pl.BlockSpec((BT2, 2), lambda i: (i + NBT, 0)),
            pl.BlockSpec((1, 1, BT2), lambda i: (i, 0, 0)),
            pl.BlockSpec((1, 1, BT2), lambda i: (i + NBT, 0, 0)),
            pl.BlockSpec((D, HID), lambda i: (0, 0)),
            pl.BlockSpec((1, HID), lambda i: (0, 0)),
            pl.BlockSpec((1, HID), lambda i: (0, 0)),
            pl.BlockSpec((HID, NREL), lambda i: (0, 0)),
            pl.BlockSpec((1, NREL), lambda i: (0, 0)),
            pl.BlockSpec((NREL, 1), lambda i: (0, 0)),
        ],
        out_specs=[pl.BlockSpec((BT2, 1), lambda i: (i, 0)),
                   pl.BlockSpec((BT2, 1), lambda i: (i, 0))],
        out_shape=[jax.ShapeDtypeStruct((G, 1), jnp.float32),
                   jax.ShapeDtypeStruct((G, 1), jnp.float32)],
    )(y32, concs, concs, relw, relw, W0a, wc, b0r, Wf, bf, ones5)


def kernel(x, edge_index, relations, concs, W0, b0, Wr, br):
    eye = jnp.eye(D, dtype=jnp.float32)
    x32 = _pack_table(x, eye[:, 0::2], eye[:, 1::2])
    y32 = _sc_gather_mul(x32, edge_index.reshape(2 * E))
    W0ab = W0[:D].astype(jnp.bfloat16)
    W0a = jnp.concatenate([W0ab[0::2], W0ab[1::2]], axis=0)
    wc = W0[D:D + 1, :]
    Wf = Wr[:, :, 0].T
    bf = br[:, 0][None, :]
    relw = relations.reshape(2 * NBT, 1, BT2)
    oA, oB = _tc_mlp(y32, concs, relw, W0a, wc, b0[None, :], Wf, bf)
    return jnp.concatenate([oA, oB], axis=0)


# speedup vs baseline: 1.2161x; 1.2021x over previous
import functools

import jax
import jax.numpy as jnp
from jax import lax
from jax.experimental import pallas as pl
from jax.experimental.pallas import tpu as pltpu
from jax.experimental.pallas import tpu_sc as plsc

N_NODES = 10000
E = 320000
D = 128
HID = 16
NREL = 5
G = E // 2

NSPLIT = 5
GS = G // NSPLIT

NC, NS = 2, 16
NW = NC * NS
GPS = GS // NW
CHP = 40
NCHUNK = GPS // CHP
NBUF = 5
NJ = NCHUNK // NBUF

BT2 = 4000
NBT = G // BT2
NBTS = GS // BT2

BPK = 2000

HIMASK = -65536


def _pack_table(x, se, so):
    def body(x_ref, se_ref, so_ref, o_ref):
        xe = jnp.dot(x_ref[...], se_ref[...],
                     preferred_element_type=jnp.float32)
        xo = jnp.dot(x_ref[...], so_ref[...],
                     preferred_element_type=jnp.float32)
        pe = lax.bitcast_convert_type(xe, jnp.int32)
        po = lax.bitcast_convert_type(xo, jnp.int32)
        lo = lax.shift_right_logical(pe + 0x8000, 16)
        hi = (po + 0x8000) & HIMASK
        o_ref[...] = lo | hi

    return pl.pallas_call(
        body,
        grid=(N_NODES // BPK,),
        in_specs=[
            pl.BlockSpec((BPK, D), lambda i: (i, 0)),
            pl.BlockSpec((D, D // 2), lambda i: (0, 0)),
            pl.BlockSpec((D, D // 2), lambda i: (0, 0)),
        ],
        out_specs=pl.BlockSpec((BPK, D // 2), lambda i: (i, 0)),
        out_shape=jax.ShapeDtypeStruct((N_NODES, D // 2), jnp.int32),
    )(x, se, so)


def _sc_gather_mul(x32, eflat, split):
    mesh = plsc.VectorSubcoreMesh(
        core_axis_name="c", subcore_axis_name="s", num_cores=NC, num_subcores=NS)

    @functools.partial(
        pl.kernel,
        out_type=jax.ShapeDtypeStruct((GS, D), jnp.int32),
        mesh=mesh,
        compiler_params=pltpu.CompilerParams(
            needs_layout_passes=False, use_tc_tiling_on_sc=False),
        scratch_types=[
            [pltpu.VMEM((2 * GPS,), jnp.int32) for _ in range(2)],
            [[pltpu.VMEM((2 * CHP, D // 2), jnp.int32) for _ in range(2)]
             for _ in range(NBUF)],
            [pltpu.VMEM((CHP, D), jnp.int32) for _ in range(NBUF)],
            [[pltpu.SemaphoreType.DMA for _ in range(2)]
             for _ in range(NBUF)],
            [pltpu.SemaphoreType.DMA for _ in range(NBUF)],
        ],
    )
    def k(x_hbm, ei_hbm, y_hbm, idx, xb, yb, smg, sst):
        wid = lax.axis_index("s") * NC + lax.axis_index("c")
        lbase = pl.multiple_of(wid * GPS, GPS)
        for side in range(2):
            pltpu.sync_copy(
                ei_hbm.at[pl.ds(2 * (side * G + split * GS) + 2 * lbase,
                                2 * GPS)], idx[side])

        def fire(ci, b):
            off = pl.multiple_of(2 * ci * CHP, 2 * CHP)
            for side in range(2):
                pltpu.async_copy(
                    x_hbm.at[idx[side].at[pl.ds(off, 2 * CHP)]],
                    xb[b][side], smg[b][side])

        def gwait(b):
            for side in range(2):
                pltpu.make_async_copy(
                    x_hbm.at[idx[side].at[pl.ds(0, 2 * CHP)]], xb[b][side],
                    smg[b][side]).wait()

        def swait(b):
            pltpu.make_async_copy(yb[b], y_hbm.at[pl.ds(0, CHP)],
                                  sst[b]).wait()

        for b in range(NBUF - 1):
            fire(b, b)

        def outer(j, carry):
            for b in range(NBUF):
                ci = j * NBUF + b
                gwait(b)

                def rowmul(r2, c2):
                    for side in range(2):
                        for kk in range(D // 32):
                            s = pl.ds(kk * 16, 16)
                            vi = xb[b][side][2 * r2, s]
                            vj = xb[b][side][2 * r2 + 1, s]
                            a0 = plsc.bitcast(vi << 16, jnp.float32)
                            b0 = plsc.bitcast(vj << 16, jnp.float32)
                            a1 = plsc.bitcast(vi & HIMASK, jnp.float32)
                            b1 = plsc.bitcast(vj & HIMASK, jnp.float32)
                            p0 = plsc.bitcast(a0 * b0, jnp.int32)
                            p1 = plsc.bitcast(a1 * b1, jnp.int32)
                            q0 = lax.shift_right_logical(p0 + 0x8000, 16)
                            q1 = (p1 + 0x8000) & HIMASK
                            yb[b][r2, pl.ds(side * 64 + kk * 16, 16)] = (
                                q0 | q1)
                    return c2

                lax.fori_loop(0, CHP, rowmul, 0, unroll=2)
                off = pl.multiple_of(ci * CHP, CHP)
                pltpu.async_copy(yb[b], y_hbm.at[pl.ds(lbase + off, CHP)],
                                 sst[b])
                nb = (b + NBUF - 1) % NBUF
                if b == 0:
                    @pl.when(j > 0)
                    def _():
                        swait(nb)
                    fire(ci + NBUF - 1, nb)
                else:
                    @pl.when(j < NJ - 1)
                    def _():
                        swait(nb)
                        fire(ci + NBUF - 1, nb)
            return carry

        lax.fori_loop(0, NJ, outer, 0)
        for b in range(NBUF):
            swait(b)

    return k(x32, eflat)


def _tc_mlp(y32, concs, relw, W0a, wc, b0r, Wf, bf, split):
    def body(y_ref, cA_ref, cB_ref, rA_ref, rB_ref, W0a_ref, wc_ref, b0_ref,
             Wf_ref, bf_ref, ones_ref, oA_ref, oB_ref):
        v = y_ref[...]
        ylo = lax.bitcast_convert_type(v << 16, jnp.float32)
        yhi = lax.bitcast_convert_type(v & HIMASK, jnp.float32)
        ylo = ylo.astype(jnp.bfloat16)
        yhi = yhi.astype(jnp.bfloat16)
        for half, c_ref, r_ref, o_ref in ((0, cA_ref, rA_ref, oA_ref),
                                          (1, cB_ref, rB_ref, oB_ref)):
            sl = slice(half * 64, half * 64 + 64)
            yh = jnp.concatenate([ylo[:, sl], yhi[:, sl]], axis=1)
            cc = c_ref[...]
            c = cc[:, 0:1] * cc[:, 1:2]
            h = jnp.dot(yh, W0a_ref[...], preferred_element_type=jnp.float32)
            h = jnp.maximum(h + c * wc_ref[...] + b0_ref[...], 0.0)
            o5 = jnp.dot(h, Wf_ref[...], preferred_element_type=jnp.float32)
            o5 = o5 + bf_ref[...]
            rel = jnp.transpose(r_ref[0])
            onehot = (rel
                      == lax.broadcasted_iota(jnp.int32, (1, NREL), 1))
            sel = o5 * onehot.astype(jnp.float32)
            o_ref[...] = jnp.dot(sel, ones_ref[...],
                                 preferred_element_type=jnp.float32)

    ones5 = jnp.ones((NREL, 1), jnp.float32)
    sb = split * NBTS
    return pl.pallas_call(
        body,
        grid=(NBTS,),
        in_specs=[
            pl.BlockSpec((BT2, D), lambda i: (i, 0)),
            pl.BlockSpec((BT2, 2), lambda i: (i + sb, 0)),
            pl.BlockSpec((BT2, 2), lambda i: (i + sb + NBT, 0)),
            pl.BlockSpec((1, 1, BT2), lambda i: (i + sb, 0, 0)),
            pl.BlockSpec((1, 1, BT2), lambda i: (i + sb + NBT, 0, 0)),
            pl.BlockSpec((D, HID), lambda i: (0, 0)),
            pl.BlockSpec((1, HID), lambda i: (0, 0)),
            pl.BlockSpec((1, HID), lambda i: (0, 0)),
            pl.BlockSpec((HID, NREL), lambda i: (0, 0)),
            pl.BlockSpec((1, NREL), lambda i: (0, 0)),
            pl.BlockSpec((NREL, 1), lambda i: (0, 0)),
        ],
        out_specs=[pl.BlockSpec((BT2, 1), lambda i: (i, 0)),
                   pl.BlockSpec((BT2, 1), lambda i: (i, 0))],
        out_shape=[jax.ShapeDtypeStruct((GS, 1), jnp.float32),
                   jax.ShapeDtypeStruct((GS, 1), jnp.float32)],
    )(y32, concs, concs, relw, relw, W0a, wc, b0r, Wf, bf, ones5)


def kernel(x, edge_index, relations, concs, W0, b0, Wr, br):
    eye = jnp.eye(D, dtype=jnp.float32)
    x32 = _pack_table(x, eye[:, 0::2], eye[:, 1::2])
    eflat = edge_index.reshape(2 * E)
    W0ab = W0[:D].astype(jnp.bfloat16)
    W0a = jnp.concatenate([W0ab[0::2], W0ab[1::2]], axis=0)
    wc = W0[D:D + 1, :]
    Wf = Wr[:, :, 0].T
    bf = br[:, 0][None, :]
    relw = relations.reshape(2 * NBT, 1, BT2)
    outsA, outsB = [], []
    for s in range(NSPLIT):
        y32 = _sc_gather_mul(x32, eflat, s)
        oA, oB = _tc_mlp(y32, concs, relw, W0a, wc, b0[None, :], Wf, bf, s)
        outsA.append(oA)
        outsB.append(oB)
    return jnp.concatenate(outsA + outsB, axis=0)


# speedup vs baseline: 1.2315x; 1.0126x over previous
import functools

import jax
import jax.numpy as jnp
from jax import lax
from jax.experimental import pallas as pl
from jax.experimental.pallas import tpu as pltpu
from jax.experimental.pallas import tpu_sc as plsc

N_NODES = 10000
E = 320000
D = 128
HID = 16
NREL = 5

NC, NS = 2, 16
NW = NC * NS
EPW = E // NW
CH = 80
NCHUNK = EPW // CH
NBUF = 5
NJ = NCHUNK // NBUF

BT2 = 4000
NBT = E // 2 // BT2

HIMASK = -65536


def _sc_gather_mul(x, row, col):
    mesh = plsc.VectorSubcoreMesh(
        core_axis_name="c", subcore_axis_name="s", num_cores=NC, num_subcores=NS)

    @functools.partial(
        pl.kernel,
        out_type=jax.ShapeDtypeStruct((E // 2, D), jnp.int32),
        mesh=mesh,
        compiler_params=pltpu.CompilerParams(
            needs_layout_passes=False, use_tc_tiling_on_sc=False),
        scratch_types=[
            pltpu.VMEM((EPW,), jnp.int32),
            pltpu.VMEM((EPW,), jnp.int32),
            [pltpu.VMEM((CH, D // 2), jnp.int32) for _ in range(NBUF)],
            [pltpu.VMEM((CH, D // 2), jnp.int32) for _ in range(NBUF)],
            [pltpu.VMEM((CH // 2, D), jnp.int32) for _ in range(NBUF)],
            [pltpu.SemaphoreType.DMA for _ in range(NBUF)],
            [pltpu.SemaphoreType.DMA for _ in range(NBUF)],
            [pltpu.SemaphoreType.DMA for _ in range(NBUF)],
        ],
    )
    def k(x_hbm, row_hbm, col_hbm, y_hbm, idxr, idxc, xi, xj, yb, smr, smc,
          sst):
        wid = lax.axis_index("s") * NC + lax.axis_index("c")
        base = pl.multiple_of(wid * EPW, EPW)
        pltpu.sync_copy(row_hbm.at[pl.ds(base, EPW)], idxr)
        pltpu.sync_copy(col_hbm.at[pl.ds(base, EPW)], idxc)

        def fire(ci, b):
            off = pl.multiple_of(ci * CH, CH)
            pltpu.async_copy(x_hbm.at[idxr.at[pl.ds(off, CH)]], xi[b], smr[b])
            pltpu.async_copy(x_hbm.at[idxc.at[pl.ds(off, CH)]], xj[b], smc[b])

        def gwait(b):
            pltpu.make_async_copy(x_hbm.at[idxr.at[pl.ds(0, CH)]], xi[b],
                                  smr[b]).wait()
            pltpu.make_async_copy(x_hbm.at[idxc.at[pl.ds(0, CH)]], xj[b],
                                  smc[b]).wait()

        def swait(b):
            pltpu.make_async_copy(yb[b], y_hbm.at[pl.ds(0, CH // 2)],
                                  sst[b]).wait()

        for b in range(NBUF - 1):
            fire(b, b)

        def outer(j, carry):
            for b in range(NBUF):
                ci = j * NBUF + b
                gwait(b)

                def rowmul(r2, c2):
                    for half in range(2):
                        r = r2 * 2 + half
                        for kk in range(D // 32):
                            s = pl.ds(kk * 16, 16)
                            vi = xi[b][r, s]
                            vj = xj[b][r, s]
                            a0 = plsc.bitcast(vi << 16, jnp.float32)
                            b0 = plsc.bitcast(vj << 16, jnp.float32)
                            a1 = plsc.bitcast(vi & HIMASK, jnp.float32)
                            b1 = plsc.bitcast(vj & HIMASK, jnp.float32)
                            p0 = plsc.bitcast(a0 * b0, jnp.int32)
                            p1 = plsc.bitcast(a1 * b1, jnp.int32)
                            q0 = lax.shift_right_logical(p0 + 0x8000, 16)
                            q1 = (p1 + 0x8000) & HIMASK
                            yb[b][r2, pl.ds(half * 64 + kk * 16, 16)] = (
                                q0 | q1)
                    return c2

                lax.fori_loop(0, CH // 2, rowmul, 0, unroll=2)
                off = pl.multiple_of(ci * CH, CH)
                pltpu.async_copy(yb[b],
                                 y_hbm.at[pl.ds((base + off) // 2, CH // 2)],
                                 sst[b])
                nb = (b + NBUF - 1) % NBUF
                if b == 0:
                    @pl.when(j > 0)
                    def _():
                        swait(nb)
                    fire(ci + NBUF - 1, nb)
                else:
                    @pl.when(j < NJ - 1)
                    def _():
                        swait(nb)
                        fire(ci + NBUF - 1, nb)
            return carry

        lax.fori_loop(0, NJ, outer, 0)
        for b in range(NBUF):
            swait(b)

    return k(x, row, col)


def _tc_mlp(y32, c4, rel2, W0a, wc, b0r, Wf, bf):
    def body(y_ref, c4_ref, rel_ref, W0a_ref, wc_ref, b0_ref, Wf_ref,
             bf_ref, ones_ref, o_ref):
        v = y_ref[...]
        ylo = lax.bitcast_convert_type(v << 16, jnp.float32)
        yhi = lax.bitcast_convert_type(v & HIMASK, jnp.float32)
        ylo = ylo.astype(jnp.bfloat16)
        yhi = yhi.astype(jnp.bfloat16)
        cc = c4_ref[...]
        rel = rel_ref[...]
        outs = []
        for half in range(2):
            sl = slice(half * 64, half * 64 + 64)
            yb = jnp.concatenate([ylo[:, sl], yhi[:, sl]], axis=1)
            c = cc[:, 2 * half:2 * half + 1] * cc[:, 2 * half + 1:2 * half + 2]
            h = jnp.dot(yb, W0a_ref[...], preferred_element_type=jnp.float32)
            h = jnp.maximum(h + c * wc_ref[...] + b0_ref[...], 0.0)
            o5 = jnp.dot(h, Wf_ref[...], preferred_element_type=jnp.float32)
            o5 = o5 + bf_ref[...]
            onehot = (rel[:, half:half + 1]
                      == lax.broadcasted_iota(jnp.int32, (1, NREL), 1))
            sel = o5 * onehot.astype(jnp.float32)
            outs.append(jnp.dot(sel, ones_ref[...],
                                preferred_element_type=jnp.float32))
        o_ref[...] = jnp.concatenate(outs, axis=1)

    ones5 = jnp.ones((NREL, 1), jnp.float32)
    return pl.pallas_call(
        body,
        grid=(NBT,),
        in_specs=[
            pl.BlockSpec((BT2, D), lambda i: (i, 0)),
            pl.BlockSpec((BT2, 4), lambda i: (i, 0)),
            pl.BlockSpec((BT2, 2), lambda i: (i, 0)),
            pl.BlockSpec((D, HID), lambda i: (0, 0)),
            pl.BlockSpec((1, HID), lambda i: (0, 0)),
            pl.BlockSpec((1, HID), lambda i: (0, 0)),
            pl.BlockSpec((HID, NREL), lambda i: (0, 0)),
            pl.BlockSpec((1, NREL), lambda i: (0, 0)),
            pl.BlockSpec((NREL, 1), lambda i: (0, 0)),
        ],
        out_specs=pl.BlockSpec((BT2, 2), lambda i: (i, 0)),
        out_shape=jax.ShapeDtypeStruct((E // 2, 2), jnp.float32),
    )(y32, c4, rel2, W0a, wc, b0r, Wf, bf, ones5)


def kernel(x, edge_index, relations, concs, W0, b0, Wr, br):
    row = edge_index[:, 0]
    col = edge_index[:, 1]
    x32 = lax.bitcast_convert_type(
        x.astype(jnp.bfloat16).reshape(N_NODES, D // 2, 2), jnp.int32)
    y32 = _sc_gather_mul(x32, row, col)
    W0ab = W0[:D].astype(jnp.bfloat16)
    W0a = jnp.concatenate([W0ab[0::2], W0ab[1::2]], axis=0)
    wc = W0[D:D + 1, :]
    Wf = Wr[:, :, 0].T
    bf = br[:, 0][None, :]
    out2 = _tc_mlp(y32, concs.reshape(E // 2, 4), relations.reshape(E // 2, 2),
                   W0a, wc, b0[None, :], Wf, bf)
    return out2.reshape(E, 1)
